# baseline (speedup 1.0000x reference)
import jax
import jax.numpy as jnp
from jax.experimental import pallas as pl
from jax.experimental.pallas import tpu as pltpu

_P = 16
_TS = 12
_L = 64
_K = 4096
_EOT = 2
_EOI = 32001
_OFFSET = 32002
_NB = 4


def _vq_kernel(x_ref, wm_ref, wp_ref, bp_ref, wproj_ref, cb_ref, flag_ref,
               out_ref, wc_s, sbb_s, cbn_s, cn2_s):
    @pl.when(pl.program_id(0) == 0)
    def _():
        wc_s[...] = jax.lax.dot_general(
            wp_ref[...], wproj_ref[...], (((1,), (0,)), ((), ())))
        bb = jax.lax.dot_general(
            bp_ref[...], wproj_ref[...], (((1,), (0,)), ((), ())))
        ones_p = jnp.ones((1, wm_ref.shape[0]), jnp.float32)
        s_col = jax.lax.dot_general(
            wm_ref[...], ones_p, (((0,), (1,)), ((), ())))
        sbb_s[...] = s_col * bb
        cb = cb_ref[...]
        nrm = jnp.sqrt(jnp.sum(cb * cb, axis=1, keepdims=True))
        cbn = cb / (nrm + 1e-6)
        cbn_s[...] = cbn
        ones_t = jnp.ones((1, cb.shape[1]), jnp.float32)
        cn2_s[...] = jax.lax.dot_general(
            ones_t, cbn * cbn, (((1,), (1,)), ((), ())))

    zs = []
    for j in range(_NB):
        xj = x_ref[j]
        mj = jax.lax.dot_general(
            wm_ref[...], xj, (((0,), (0,)), ((), ())))
        zs.append(jax.lax.dot_general(
            mj, wc_s[...], (((1,), (0,)), ((), ()))) + sbb_s[...])
    z = jnp.concatenate(zs, axis=0)
    zn = z / (jnp.sqrt(jnp.sum(z * z, axis=1, keepdims=True)) + 1e-6)
    dots = jax.lax.dot_general(zn, cbn_s[...], (((1,), (1,)), ((), ())))
    scores = cn2_s[...] - 2.0 * dots
    idx = jnp.argmin(scores, axis=1).astype(jnp.int32)
    flag = flag_ref[0]
    rows = jnp.concatenate(
        [idx.reshape(_NB, _L) + _OFFSET,
         jnp.full((_NB, 1), _EOI, jnp.int32),
         flag * jnp.broadcast_to(
             jax.lax.broadcasted_iota(jnp.int32, (1, 2), 1) + _EOT - 1,
             (_NB, 2))],
        axis=1)
    out_ref[...] = rows.reshape(_NB, 1, _L + 3)


def kernel(image, append_empty_text, W_patch, b_patch, W_mix, W_proj, codebook):
    B, C, H, _ = image.shape
    G = H // _P
    NP = G * G
    D = W_patch.shape[1]
    flag = jnp.asarray(append_empty_text).astype(jnp.int32).reshape(1)
    bp2 = b_patch.reshape(1, D)

    def vq_call(xc):
        nb = xc.shape[0]
        return pl.pallas_call(
            _vq_kernel,
            grid=(nb // _NB,),
            in_specs=[
                pl.BlockSpec((_NB, NP, C * _P * _P), lambda b: (b, 0, 0)),
                pl.BlockSpec((NP, _L), lambda b: (0, 0)),
                pl.BlockSpec((C * _P * _P, D), lambda b: (0, 0)),
                pl.BlockSpec((1, D), lambda b: (0, 0)),
                pl.BlockSpec((D, _TS), lambda b: (0, 0)),
                pl.BlockSpec((_K, _TS), lambda b: (0, 0)),
                pl.BlockSpec(memory_space=pltpu.SMEM),
            ],
            out_specs=pl.BlockSpec((_NB, 1, _L + 3), lambda b: (b, 0, 0)),
            out_shape=jax.ShapeDtypeStruct((nb, 1, _L + 3), jnp.int32),
            scratch_shapes=[
                pltpu.VMEM((C * _P * _P, _TS), jnp.float32),
                pltpu.VMEM((_L, _TS), jnp.float32),
                pltpu.VMEM((_K, _TS), jnp.float32),
                pltpu.VMEM((1, _K), jnp.float32),
            ],
            compiler_params=pltpu.CompilerParams(
                dimension_semantics=("arbitrary",)),
        )(xc, W_mix, W_patch, bp2, W_proj, codebook, flag)

    n_chunks = 4
    cb_sz = B // n_chunks
    outs = []
    for i in range(n_chunks):
        img_c = image[i * cb_sz:(i + 1) * cb_sz]
        xc = img_c.reshape(cb_sz, C, G, _P, G, _P)
        xc = xc.transpose(0, 2, 4, 1, 3, 5).reshape(cb_sz, NP, C * _P * _P)
        outs.append(vq_call(xc))
    return jnp.concatenate(outs, axis=0).reshape(B, _L + 3)


# speedup vs baseline: 4.9092x; 4.9092x over previous
import jax
import jax.numpy as jnp
from jax.experimental import pallas as pl
from jax.experimental.pallas import tpu as pltpu

_P = 16
_G = 16
_TS = 12
_L = 64
_K = 4096
_EOT = 2
_EOI = 32001
_OFFSET = 32002
_NBX = 16


def _vq_kernel(x_ref, wm_ref, wm2_ref, wp_ref, bp_ref, wproj_ref, cb_ref,
               flag_ref, out_ref, wbig_s, u_s, mask_s, s1_s, z_s, sbb_s,
               cbn_s, cn2_s):
    C = x_ref.shape[1]
    NP = _G * _G
    PP = _P * _P
    GT = _G * _TS

    @pl.when(pl.program_id(0) == 0)
    def _():
        wc = jax.lax.dot_general(
            wp_ref[...], wproj_ref[...], (((1,), (0,)), ((), ())))
        w3row = jax.lax.broadcasted_iota(jnp.int32, (PP * _G, 1), 0)
        rowgw = (w3row // _P) % _G
        rowpypx = (w3row // (PP)) * _P + w3row % _P
        p3 = (rowpypx == jax.lax.broadcasted_iota(
            jnp.int32, (1, PP), 1)).astype(jnp.float32)
        lanegw = jax.lax.broadcasted_iota(jnp.int32, (1, GT), 1) % _G
        lanet = jax.lax.broadcasted_iota(jnp.int32, (1, GT), 1) // _G
        e2 = (jax.lax.broadcasted_iota(jnp.int32, (_TS, 1), 0) ==
              lanet).astype(jnp.float32)
        dmask = (rowgw == lanegw).astype(jnp.float32)
        for c in range(C):
            v2 = jax.lax.dot_general(
                p3, wc[c * PP:(c + 1) * PP, :],
                (((1,), (0,)), ((), ())))
            vrep = jax.lax.dot_general(
                v2, e2, (((1,), (0,)), ((), ())))
            wbig_s[c] = vrep * dmask
        u0 = wm2_ref[...]
        for c in range(C):
            u_s[c * _G:(c + 1) * _G, :] = u0
        rg = jax.lax.broadcasted_iota(jnp.int32, (_G * _L, 1), 0) // _L
        lg = jax.lax.broadcasted_iota(jnp.int32, (1, GT), 1) % _G
        mask_s[...] = (rg == lg).astype(jnp.float32)
        bb = jax.lax.dot_general(
            bp_ref[...], wproj_ref[...], (((1,), (0,)), ((), ())))
        ones_p = jnp.ones((1, NP), jnp.float32)
        s_col = jax.lax.dot_general(
            wm_ref[...], ones_p, (((0,), (1,)), ((), ())))
        sbb_s[...] = s_col * bb
        cb = cb_ref[...]
        nrm = jnp.sqrt(jnp.sum(cb * cb, axis=1, keepdims=True))
        cbn = cb / (nrm + 1e-6)
        cbn_s[...] = cbn
        ones_t = jnp.ones((1, cb.shape[1]), jnp.float32)
        cn2_s[...] = jax.lax.dot_general(
            ones_t, cbn * cbn, (((1,), (1,)), ((), ())))

    for c in range(C):
        op = x_ref[:, c].reshape(_NBX * _G, PP * _G)
        s1_s[c] = jax.lax.dot_general(
            op, wbig_s[c], (((1,), (0,)), ((), ())))

    lanet = jax.lax.broadcasted_iota(jnp.int32, (1, GT), 1) // _G
    tsum = (jax.lax.broadcasted_iota(jnp.int32, (_TS, 1), 0) ==
            lanet).astype(jnp.float32)

    def mix_one(j, _):
        bj = jnp.concatenate(
            [s1_s[c, pl.ds(j * _G, _G), :] for c in range(C)], axis=0)
        cj = jax.lax.dot_general(
            u_s[...], bj, (((0,), (0,)), ((), ())))
        masked = cj * mask_s[...]
        r = jnp.sum(masked.reshape(_G, _L, GT), axis=0)
        zj = jax.lax.dot_general(
            r, tsum, (((1,), (1,)), ((), ()))) + sbb_s[...]
        z_s[pl.ds(j * _L, _L), :] = zj
        return 0

    jax.lax.fori_loop(0, _NBX, mix_one, 0)

    flag = flag_ref[0]
    for jj in range(_NBX // 4):
        z = z_s[pl.ds(jj * 4 * _L, 4 * _L), :]
        zn = z / (jnp.sqrt(jnp.sum(z * z, axis=1, keepdims=True)) + 1e-6)
        dots = jax.lax.dot_general(zn, cbn_s[...], (((1,), (1,)), ((), ())))
        scores = cn2_s[...] - 2.0 * dots
        idx = jnp.argmin(scores, axis=1).astype(jnp.int32)
        rows = jnp.concatenate(
            [idx.reshape(4, _L) + _OFFSET,
             jnp.full((4, 1), _EOI, jnp.int32),
             flag * jnp.broadcast_to(
                 jax.lax.broadcasted_iota(jnp.int32, (1, 2), 1) + _EOT - 1,
                 (4, 2))],
            axis=1)
        out_ref[jj * 4:(jj + 1) * 4] = rows.reshape(4, 1, _L + 3)


def kernel(image, append_empty_text, W_patch, b_patch, W_mix, W_proj, codebook):
    B, C, H, _ = image.shape
    NP = _G * _G
    PP = _P * _P
    D = W_patch.shape[1]
    GT = _G * _TS
    x4 = image.reshape(B, C, _G, _P * H)
    wmix2 = W_mix.reshape(_G, _G * _L)
    flag = jnp.asarray(append_empty_text).astype(jnp.int32).reshape(1)

    out = pl.pallas_call(
        _vq_kernel,
        grid=(B // _NBX,),
        in_specs=[
            pl.BlockSpec((_NBX, C, _G, _P * H), lambda b: (b, 0, 0, 0)),
            pl.BlockSpec((NP, _L), lambda b: (0, 0)),
            pl.BlockSpec((_G, _G * _L), lambda b: (0, 0)),
            pl.BlockSpec((C * PP, D), lambda b: (0, 0)),
            pl.BlockSpec((1, D), lambda b: (0, 0)),
            pl.BlockSpec((D, _TS), lambda b: (0, 0)),
            pl.BlockSpec((_K, _TS), lambda b: (0, 0)),
            pl.BlockSpec(memory_space=pltpu.SMEM),
        ],
        out_specs=pl.BlockSpec((_NBX, 1, _L + 3), lambda b: (b, 0, 0)),
        out_shape=jax.ShapeDtypeStruct((B, 1, _L + 3), jnp.int32),
        scratch_shapes=[
            pltpu.VMEM((C, PP * _G, GT), jnp.float32),
            pltpu.VMEM((C * _G, _G * _L), jnp.float32),
            pltpu.VMEM((_G * _L, GT), jnp.float32),
            pltpu.VMEM((C, _NBX * _G, GT), jnp.float32),
            pltpu.VMEM((_NBX * _L, _TS), jnp.float32),
            pltpu.VMEM((_L, _TS), jnp.float32),
            pltpu.VMEM((_K, _TS), jnp.float32),
            pltpu.VMEM((1, _K), jnp.float32),
        ],
        compiler_params=pltpu.CompilerParams(
            dimension_semantics=("arbitrary",)),
    )(x4, W_mix, wmix2, W_patch, b_patch.reshape(1, D), W_proj, codebook,
      flag)
    return out.reshape(B, _L + 3)
